# baseline (device time: 7664 ns/iter reference)
import jax
import jax.numpy as jnp
from jax import lax
from jax.experimental import pallas as pl
from jax.experimental.pallas import tpu as pltpu

_N_CHUNKS = 4


def kernel(x, pi):
    rows = x.shape[1]
    chunk = rows // _N_CHUNKS

    def body(x_ref, pi_ref, out_ref, comm_ref, send_sems, recv_sems):
        my_x = lax.axis_index("x")
        my_y = lax.axis_index("y")
        my_z = lax.axis_index("z")
        swap = pi_ref[0] != 0

        @pl.when(jnp.logical_not(swap))
        def _():
            out_ref[...] = x_ref[...].astype(out_ref.dtype)

        @pl.when(swap)
        def _():
            neighbor = (my_x, 1 - my_y, my_z)
            barrier_sem = pltpu.get_barrier_semaphore()
            pl.semaphore_signal(
                barrier_sem, inc=1,
                device_id=neighbor, device_id_type=pl.DeviceIdType.MESH,
            )
            rdmas = []
            for c in range(_N_CHUNKS):
                lo, hi = c * chunk, (c + 1) * chunk
                comm_ref[lo:hi, :] = x_ref[0, lo:hi, :].astype(comm_ref.dtype)
                if c == 0:
                    pl.semaphore_wait(barrier_sem, 1)
                rdma = pltpu.make_async_remote_copy(
                    src_ref=comm_ref.at[pl.ds(lo, chunk), :],
                    dst_ref=out_ref.at[0, pl.ds(lo, chunk), :],
                    send_sem=send_sems.at[c],
                    recv_sem=recv_sems.at[c],
                    device_id=neighbor,
                    device_id_type=pl.DeviceIdType.MESH,
                )
                rdma.start()
                rdmas.append(rdma)
            for rdma in rdmas:
                rdma.wait()

    return pl.pallas_call(
        body,
        out_shape=jax.ShapeDtypeStruct(x.shape, jnp.bfloat16),
        in_specs=[
            pl.BlockSpec(memory_space=pltpu.VMEM),
            pl.BlockSpec(memory_space=pltpu.SMEM),
        ],
        out_specs=pl.BlockSpec(memory_space=pltpu.VMEM),
        scratch_shapes=[
            pltpu.VMEM((rows, x.shape[2]), jnp.bfloat16),
            pltpu.SemaphoreType.DMA((_N_CHUNKS,)),
            pltpu.SemaphoreType.DMA((_N_CHUNKS,)),
        ],
        compiler_params=pltpu.CompilerParams(collective_id=0),
    )(x, pi)


# device time: 7607 ns/iter; 1.0075x vs baseline; 1.0075x over previous
import jax
import jax.numpy as jnp
from jax import lax
from jax.experimental import pallas as pl
from jax.experimental.pallas import tpu as pltpu


def kernel(x, pi):
    def body(x_ref, pi_ref, out_ref, comm_ref, send_sem, recv_sem):
        my_x = lax.axis_index("x")
        my_y = lax.axis_index("y")
        my_z = lax.axis_index("z")
        swap = pi_ref[0] != 0

        @pl.when(jnp.logical_not(swap))
        def _():
            out_ref[...] = x_ref[...].astype(out_ref.dtype)

        @pl.when(swap)
        def _():
            neighbor = (my_x, 1 - my_y, my_z)
            barrier_sem = pltpu.get_barrier_semaphore()
            pl.semaphore_signal(
                barrier_sem, inc=1,
                device_id=neighbor, device_id_type=pl.DeviceIdType.MESH,
            )
            comm_ref[...] = x_ref[...].astype(comm_ref.dtype)
            pl.semaphore_wait(barrier_sem, 1)

            rdma = pltpu.make_async_remote_copy(
                src_ref=comm_ref,
                dst_ref=out_ref,
                send_sem=send_sem,
                recv_sem=recv_sem,
                device_id=neighbor,
                device_id_type=pl.DeviceIdType.MESH,
            )
            rdma.start()
            rdma.wait()

    return pl.pallas_call(
        body,
        out_shape=jax.ShapeDtypeStruct(x.shape, jnp.bfloat16),
        in_specs=[
            pl.BlockSpec(memory_space=pltpu.VMEM),
            pl.BlockSpec(memory_space=pltpu.SMEM),
        ],
        out_specs=pl.BlockSpec(memory_space=pltpu.VMEM),
        scratch_shapes=[
            pltpu.VMEM(x.shape, jnp.bfloat16),
            pltpu.SemaphoreType.DMA,
            pltpu.SemaphoreType.DMA,
        ],
        compiler_params=pltpu.CompilerParams(collective_id=0),
    )(x, pi)
